# baseline (speedup 1.0000x reference)
import jax
import jax.numpy as jnp
from jax import lax
from jax.experimental import pallas as pl
from jax.experimental.pallas import tpu as pltpu
from jax.experimental.pallas import tpu_sc as plsc

_N_T = 2048
_N_S = 2048
_D = 16
_NC = 2
_NSUB = 16
_NW = _NC * _NSUB
_ROWS_PER_W = _N_T // _NW
_CH = 8
_TR = 2


def _lane_sums(scr_ref, accs):
    for d in range(_D):
        scr_ref[d, :] = accs[d]
    row_idx = lax.iota(jnp.int32, 16)
    out = jnp.zeros((_D,), jnp.float32)
    for l in range(16):
        col_idx = jnp.full((16,), l, jnp.int32)
        out = out + plsc.load_gather(scr_ref, [row_idx, col_idx])
    return out


def _sc_body(xt_hbm, a_hbm, out_hbm, xt_v, a_v, o_v, scr_v, sem):
    wid = lax.axis_index("s") * _NC + lax.axis_index("c")
    base = wid * _ROWS_PER_W
    pltpu.sync_copy(xt_hbm, xt_v)

    def chunk_body(ci, carry):
        pltpu.sync_copy(a_hbm.at[pl.ds(base + ci * _CH, _CH), :], a_v)
        for t0 in range(0, _CH, _TR):
            def s_body(c, accs):
                va0 = a_v[t0, pl.ds(c * 16, 16)]
                va1 = a_v[t0 + 1, pl.ds(c * 16, 16)]
                new = []
                for d in range(_D):
                    xt = xt_v[d, pl.ds(c * 16, 16)]
                    new.append(accs[2 * d] + va0 * xt)
                    new.append(accs[2 * d + 1] + va1 * xt)
                return tuple(new)

            accs = tuple(jnp.zeros((16,), jnp.float32)
                         for _ in range(2 * _D))
            accs = lax.fori_loop(0, _N_S // 16, s_body, accs)
            o_v[t0, :] = _lane_sums(scr_v, [accs[2 * d] for d in range(_D)])
            o_v[t0 + 1, :] = _lane_sums(
                scr_v, [accs[2 * d + 1] for d in range(_D)])
        pltpu.sync_copy(o_v, out_hbm.at[pl.ds(base + ci * _CH, _CH), :])
        return carry

    lax.fori_loop(0, _ROWS_PER_W // _CH, chunk_body, 0)


def kernel(x, a):
    f = pl.kernel(
        _sc_body,
        mesh=plsc.VectorSubcoreMesh(core_axis_name="c", subcore_axis_name="s"),
        out_type=jax.ShapeDtypeStruct((_N_T, _D), jnp.float32),
        scratch_types=[
            pltpu.VMEM((_D, _N_S), jnp.float32),
            pltpu.VMEM((_CH, _N_S), jnp.float32),
            pltpu.VMEM((_CH, _D), jnp.float32),
            pltpu.VMEM((16, 16), jnp.float32),
            pltpu.SemaphoreType.DMA,
        ],
        compiler_params=pltpu.CompilerParams(needs_layout_passes=False),
    )
    return f(x.T.reshape(_D, _N_S), a)


# speedup vs baseline: 2.4993x; 2.4993x over previous
import jax
import jax.numpy as jnp
from jax import lax
from jax.experimental import pallas as pl
from jax.experimental.pallas import tpu as pltpu
from jax.experimental.pallas import tpu_sc as plsc

_N_T = 2048
_N_S = 2048
_D = 16
_NC = 2
_NSUB = 16
_NW = _NC * _NSUB
_SC_ROWS = 256
_RPW = _SC_ROWS // _NW
_TR = 2


def _lane_sums(scr_ref, accs):
    for d in range(_D):
        scr_ref[d, :] = accs[d]
    row_idx = lax.iota(jnp.int32, 16)
    out = jnp.zeros((_D,), jnp.float32)
    for l in range(16):
        col_idx = jnp.full((16,), l, jnp.int32)
        out = out + plsc.load_gather(scr_ref, [row_idx, col_idx])
    return out


def _sc_body(xt_hbm, a_hbm, out_hbm, xt_v, a_v, o_v, scr_v, sem):
    wid = lax.axis_index("s") * _NC + lax.axis_index("c")
    base = wid * _RPW
    pltpu.sync_copy(xt_hbm, xt_v)
    pltpu.sync_copy(
        a_hbm.at[pl.ds((_N_T - _SC_ROWS) + base, _RPW), :], a_v)
    for t0 in range(0, _RPW, _TR):
        def s_body(c, accs):
            va0 = a_v[t0, pl.ds(c * 16, 16)]
            va1 = a_v[t0 + 1, pl.ds(c * 16, 16)]
            new = []
            for d in range(_D):
                xt = xt_v[d, pl.ds(c * 16, 16)]
                new.append(accs[2 * d] + va0 * xt)
                new.append(accs[2 * d + 1] + va1 * xt)
            return tuple(new)

        accs = tuple(jnp.zeros((16,), jnp.float32) for _ in range(2 * _D))
        accs = lax.fori_loop(0, _N_S // 16, s_body, accs)
        o_v[t0, :] = _lane_sums(scr_v, [accs[2 * d] for d in range(_D)])
        o_v[t0 + 1, :] = _lane_sums(
            scr_v, [accs[2 * d + 1] for d in range(_D)])
    pltpu.sync_copy(o_v, out_hbm.at[pl.ds(base, _RPW), :])


def _mm_kernel(a_ref, x_ref, o_ref):
    o_ref[...] = jnp.dot(a_ref[...], x_ref[...],
                         preferred_element_type=jnp.float32)


def kernel(x, a):
    n_t, n_s = a.shape
    d = x.shape[1]

    sc_f = pl.kernel(
        _sc_body,
        mesh=plsc.VectorSubcoreMesh(core_axis_name="c", subcore_axis_name="s"),
        out_type=jax.ShapeDtypeStruct((_SC_ROWS, _D), jnp.float32),
        scratch_types=[
            pltpu.VMEM((_D, _N_S), jnp.float32),
            pltpu.VMEM((_RPW, _N_S), jnp.float32),
            pltpu.VMEM((_RPW, _D), jnp.float32),
            pltpu.VMEM((16, 16), jnp.float32),
            pltpu.SemaphoreType.DMA,
        ],
        compiler_params=pltpu.CompilerParams(needs_layout_passes=False),
    )
    out_sc = sc_f(x.T.reshape(_D, _N_S), a)

    tc_rows = n_t - _SC_ROWS
    bm = 896
    out_tc = pl.pallas_call(
        _mm_kernel,
        grid=(tc_rows // bm,),
        in_specs=[
            pl.BlockSpec((bm, n_s), lambda i: (i, 0)),
            pl.BlockSpec((n_s, d), lambda i: (0, 0)),
        ],
        out_specs=pl.BlockSpec((bm, d), lambda i: (i, 0)),
        out_shape=jax.ShapeDtypeStruct((tc_rows, d), jnp.float32),
    )(a, x)
    return jnp.concatenate([out_tc, out_sc], axis=0)


# speedup vs baseline: 7.4161x; 2.9672x over previous
import jax
import jax.numpy as jnp
from jax.experimental import pallas as pl


def _mm_kernel(a_ref, x_ref, o_ref):
    @pl.when(pl.program_id(1) == 0)
    def _():
        o_ref[...] = jnp.zeros_like(o_ref)

    o_ref[...] += jnp.dot(a_ref[...], x_ref[...],
                          preferred_element_type=jnp.float32)


def kernel(x, a):
    n_t, n_s = a.shape
    d = x.shape[1]
    bm = 1024
    bk = 1024
    return pl.pallas_call(
        _mm_kernel,
        grid=(n_t // bm, n_s // bk),
        in_specs=[
            pl.BlockSpec((bm, bk), lambda i, k: (i, k)),
            pl.BlockSpec((bk, d), lambda i, k: (k, 0)),
        ],
        out_specs=pl.BlockSpec((bm, d), lambda i, k: (i, 0)),
        out_shape=jax.ShapeDtypeStruct((n_t, d), jnp.float32),
    )(a, x)


# speedup vs baseline: 7.8979x; 1.0650x over previous
import jax
import jax.numpy as jnp
from jax.experimental import pallas as pl


def _mm_kernel(a_ref, x_ref, o_ref):
    o_ref[...] = jnp.dot(a_ref[...], x_ref[...],
                         preferred_element_type=jnp.float32)


def kernel(x, a):
    n_t, n_s = a.shape
    d = x.shape[1]
    bm = 1024
    return pl.pallas_call(
        _mm_kernel,
        grid=(n_t // bm,),
        in_specs=[
            pl.BlockSpec((bm, n_s), lambda i: (i, 0)),
            pl.BlockSpec((n_s, d), lambda i: (0, 0)),
        ],
        out_specs=pl.BlockSpec((bm, d), lambda i: (i, 0)),
        out_shape=jax.ShapeDtypeStruct((n_t, d), jnp.float32),
    )(a, x)
